# baseline (speedup 1.0000x reference)
import functools

import jax
import jax.numpy as jnp
from jax import lax
from jax.experimental import pallas as pl
from jax.experimental.pallas import tpu as pltpu
from jax.experimental.pallas import tpu_sc as plsc

NC, NS = 2, 16
NW = NC * NS
B = 128
ZR = 64


def _cdiv(a, b):
    return -(-a // b)


def _mesh():
    return plsc.VectorSubcoreMesh(core_axis_name="c", subcore_axis_name="s")


_SC_PARAMS = None


def _sc_degrees(src_idx, dst_idx, ones, zeros, acc_rows, nb):
    rows_per_sub = acc_rows // NS
    n_zcopy = rows_per_sub // ZR

    out_t = [jax.ShapeDtypeStruct((NC, acc_rows, 16), jnp.float32)] * 2

    @functools.partial(
        pl.kernel,
        out_type=out_t,
        mesh=_mesh(),
        scratch_types=[
            pltpu.VMEM((nb, B), jnp.int32),
            pltpu.VMEM((nb, B), jnp.int32),
            pltpu.VMEM((B, 16), jnp.float32),
            pltpu.VMEM((ZR, 16), jnp.float32),
            pltpu.VMEM_SHARED((acc_rows, 16), jnp.float32),
            pltpu.VMEM_SHARED((acc_rows, 16), jnp.float32),
        ],
        compiler_params=pltpu.CompilerParams(use_tc_tiling_on_sc=False),
    )
    def k(src_hbm, dst_hbm, ones_hbm, zeros_hbm, osrc_hbm, odst_hbm,
          iblk_s, iblk_d, ones_v, zb, dsrc_sh, ddst_sh):
        c = lax.axis_index("c")
        s = lax.axis_index("s")
        w = s * NC + c
        pltpu.sync_copy(ones_hbm, ones_v)
        pltpu.sync_copy(zeros_hbm, zb)

        @pl.loop(0, n_zcopy)
        def _(i):
            r = s * rows_per_sub + i * ZR
            pltpu.sync_copy(zb, dsrc_sh.at[pl.ds(r, ZR)])
            pltpu.sync_copy(zb, ddst_sh.at[pl.ds(r, ZR)])

        plsc.subcore_barrier()

        pltpu.sync_copy(src_hbm.at[w], iblk_s)
        pltpu.sync_copy(dst_hbm.at[w], iblk_d)

        @pl.loop(0, nb)
        def _(j):
            pltpu.sync_copy(ones_v, dsrc_sh.at[iblk_s.at[j]], add=True)
            pltpu.sync_copy(ones_v, ddst_sh.at[iblk_d.at[j]], add=True)

        plsc.subcore_barrier()

        @pl.loop(0, n_zcopy)
        def _(i):
            r = s * rows_per_sub + i * ZR
            pltpu.sync_copy(dsrc_sh.at[pl.ds(r, ZR)], osrc_hbm.at[c, pl.ds(r, ZR)])
            pltpu.sync_copy(ddst_sh.at[pl.ds(r, ZR)], odst_hbm.at[c, pl.ds(r, ZR)])

    return k(src_idx, dst_idx, ones, zeros)


def _sc_segment_sum(h_r, src0, src1, dst_idx, zeros, acc_rows, nb):
    dh = h_r.shape[1]
    rows_per_sub = acc_rows // NS
    n_zcopy = rows_per_sub // ZR

    @functools.partial(
        pl.kernel,
        out_type=jax.ShapeDtypeStruct((NC, 2, acc_rows, dh), jnp.float32),
        mesh=_mesh(),
        scratch_types=[
            pltpu.VMEM((nb, B), jnp.int32),
            pltpu.VMEM((nb, B), jnp.int32),
            pltpu.VMEM((nb, B), jnp.int32),
            pltpu.VMEM((B, dh), jnp.float32),
            pltpu.VMEM((B, dh), jnp.float32),
            pltpu.VMEM((ZR, dh), jnp.float32),
            pltpu.VMEM_SHARED((acc_rows, dh), jnp.float32),
            pltpu.SemaphoreType.DMA,
            pltpu.SemaphoreType.DMA,
            pltpu.SemaphoreType.DMA,
            pltpu.SemaphoreType.DMA,
        ],
        compiler_params=pltpu.CompilerParams(use_tc_tiling_on_sc=False),
    )
    def k(h_hbm, s0_hbm, s1_hbm, dst_hbm, zeros_hbm, out_hbm, isv0, isv1,
          idblk, buf0, buf1, zb, acc_sh, sem0, sem1, sem2, sem3):
        c = lax.axis_index("c")
        s = lax.axis_index("s")
        w = s * NC + c
        pltpu.sync_copy(s0_hbm.at[w], isv0)
        pltpu.sync_copy(s1_hbm.at[w], isv1)
        pltpu.sync_copy(dst_hbm.at[w], idblk)
        pltpu.sync_copy(zeros_hbm, zb)

        for half, isv in ((0, isv0), (1, isv1)):
            @pl.loop(0, n_zcopy)
            def _(i):
                r = s * rows_per_sub + i * ZR
                pltpu.sync_copy(zb, acc_sh.at[pl.ds(r, ZR)])

            plsc.subcore_barrier()

            pltpu.async_copy(h_hbm.at[isv.at[0]], buf0, sem0)
            pltpu.async_copy(h_hbm.at[isv.at[1]], buf1, sem1)

            @pl.loop(0, nb // 2)
            def _(jj):
                j0 = 2 * jj
                j1 = j0 + 1
                pltpu.make_async_copy(h_hbm.at[isv.at[j0]], buf0, sem0).wait()
                pltpu.async_copy(buf0, acc_sh.at[idblk.at[j0]], sem2, add=True)
                pltpu.make_async_copy(h_hbm.at[isv.at[j1]], buf1, sem1).wait()
                pltpu.async_copy(buf1, acc_sh.at[idblk.at[j1]], sem3, add=True)

                @pl.when(jj < nb // 2 - 1)
                def _():
                    pltpu.make_async_copy(buf0, acc_sh.at[idblk.at[0]],
                                          sem2).wait()
                    pltpu.async_copy(h_hbm.at[isv.at[j0 + 2]], buf0, sem0)
                    pltpu.make_async_copy(buf1, acc_sh.at[idblk.at[0]],
                                          sem3).wait()
                    pltpu.async_copy(h_hbm.at[isv.at[j1 + 2]], buf1, sem1)

            pltpu.make_async_copy(buf0, acc_sh.at[idblk.at[0]], sem2).wait()
            pltpu.make_async_copy(buf1, acc_sh.at[idblk.at[0]], sem3).wait()

            plsc.subcore_barrier()

            @pl.loop(0, n_zcopy)
            def _(i):
                r = s * rows_per_sub + i * ZR
                pltpu.sync_copy(acc_sh.at[pl.ds(r, ZR)],
                                out_hbm.at[c, half, pl.ds(r, ZR)])

    return k(h_r, src0, src1, dst_idx, zeros)


def _tc_matmul(x, w):
    def body(x_ref, w_ref, o_ref):
        o_ref[...] = jnp.dot(x_ref[...], w_ref[...],
                             preferred_element_type=jnp.float32)

    return pl.pallas_call(
        body,
        out_shape=jax.ShapeDtypeStruct((x.shape[0], w.shape[1]), jnp.float32),
    )(x, w)


def _tc_prep(dsrc_p, ddst_p, xw1, n):
    acc_rows = dsrc_p.shape[1]
    d_h = xw1.shape[1]

    def body(ds_ref, dd_ref, xw_ref, h_ref, nrm_ref):
        deg_out = jnp.sum(ds_ref[0], axis=1) + jnp.sum(ds_ref[1], axis=1)
        deg_in = jnp.sum(dd_ref[0], axis=1) + jnp.sum(dd_ref[1], axis=1)
        norm_src = jnp.where(deg_out > 0,
                             lax.rsqrt(jnp.maximum(deg_out, 1e-12)), 0.0)
        norm_dst = jnp.where(deg_in > 0,
                             lax.rsqrt(jnp.maximum(deg_in, 1e-12)), 0.0)
        nrm_ref[0, :] = norm_src
        nrm_ref[1, :] = norm_dst
        h_ref[...] = xw_ref[...] * norm_src[:n][:, None]

    return pl.pallas_call(
        body,
        out_shape=[
            jax.ShapeDtypeStruct((n, d_h), jnp.float32),
            jax.ShapeDtypeStruct((2, acc_rows), jnp.float32),
        ],
    )(dsrc_p, ddst_p, xw1)


def _tc_mid(m_parts, nrm, b1, w2, n):
    def body(m_ref, nrm_ref, b_ref, w_ref, z_ref, h2_ref):
        mp = m_ref[0] + m_ref[1]
        m = jnp.concatenate([mp[0, :n, :], mp[1, :n, :]], axis=1)
        z = jnp.maximum(m * nrm_ref[1, :n][:, None] + b_ref[...], 0.0)
        z_ref[...] = z
        h2 = z * nrm_ref[0, :n][:, None]
        h2_ref[...] = jnp.dot(h2, w_ref[...], preferred_element_type=jnp.float32)

    d = 2 * m_parts.shape[3]
    return pl.pallas_call(
        body,
        out_shape=[
            jax.ShapeDtypeStruct((n, d), jnp.float32),
            jax.ShapeDtypeStruct((n, w2.shape[1]), jnp.float32),
        ],
    )(m_parts, nrm, b1, w2)


def _tc_final(m_parts, nrm, b2, n):
    def body(m_ref, nrm_ref, b_ref, o_ref):
        mp = m_ref[0] + m_ref[1]
        m = jnp.concatenate([mp[0, :n, :], mp[1, :n, :]], axis=1)
        o_ref[...] = m * nrm_ref[1, :n][:, None] + b_ref[...]

    d = 2 * m_parts.shape[3]
    return pl.pallas_call(
        body,
        out_shape=jax.ShapeDtypeStruct((n, d), jnp.float32),
    )(m_parts, nrm, b2)


def kernel(in_feat, edge_index, W1, b1, W2, b2):
    n, d_in = in_feat.shape
    e = edge_index.shape[1]
    dh = d_in // 2

    nb = _cdiv(e, NW * B)
    nb += nb % 2
    e_pad = NW * nb * B
    acc_rows = _cdiv(n + 1, NS * ZR) * (NS * ZR)

    src = edge_index[0].astype(jnp.int32)
    dst = edge_index[1].astype(jnp.int32)
    pad = e_pad - e
    src_g = jnp.pad(src, (0, pad)).reshape(NW, nb, B)
    src0 = src_g * 2
    src1 = src_g * 2 + 1
    src_d = jnp.pad(src, (0, pad), constant_values=n).reshape(NW, nb, B)
    dst_d = jnp.pad(dst, (0, pad), constant_values=n).reshape(NW, nb, B)

    ones16 = jnp.zeros((B, 16), jnp.float32).at[:, 0].set(1.0)
    zeros16 = jnp.zeros((ZR, 16), jnp.float32)
    zeros_dh = jnp.zeros((ZR, dh), jnp.float32)

    b1r = b1.reshape(1, -1)
    b2r = b2.reshape(1, -1)

    dsrc_p, ddst_p = _sc_degrees(src_d, dst_d, ones16, zeros16, acc_rows, nb)
    xw1 = _tc_matmul(in_feat, W1)
    h1in, nrm = _tc_prep(dsrc_p, ddst_p, xw1, n)
    m1 = _sc_segment_sum(h1in.reshape(2 * n, dh), src0, src1, dst_d,
                         zeros_dh, acc_rows, nb)
    z, h2in = _tc_mid(m1, nrm, b1r, W2, n)
    m2 = _sc_segment_sum(h2in.reshape(2 * n, dh), src0, src1, dst_d,
                         zeros_dh, acc_rows, nb)
    h2 = _tc_final(m2, nrm, b2r, n)
    return (h2, z)


# speedup vs baseline: 1.0195x; 1.0195x over previous
import functools

import jax
import jax.numpy as jnp
from jax import lax
from jax.experimental import pallas as pl
from jax.experimental.pallas import tpu as pltpu
from jax.experimental.pallas import tpu_sc as plsc

NC, NS = 2, 16
NW = NC * NS
B = 128
ZR = 64


def _cdiv(a, b):
    return -(-a // b)


def _mesh():
    return plsc.VectorSubcoreMesh(core_axis_name="c", subcore_axis_name="s")


_SC_PARAMS = None


def _sc_degrees(src_idx, dst_idx, ones, zeros, acc_rows, nb):
    rows_per_sub = acc_rows // NS
    n_zcopy = rows_per_sub // ZR

    out_t = [jax.ShapeDtypeStruct((NC, acc_rows, 16), jnp.float32)] * 2

    @functools.partial(
        pl.kernel,
        out_type=out_t,
        mesh=_mesh(),
        scratch_types=[
            pltpu.VMEM((nb, B), jnp.int32),
            pltpu.VMEM((nb, B), jnp.int32),
            pltpu.VMEM((B, 16), jnp.float32),
            pltpu.VMEM((ZR, 16), jnp.float32),
            pltpu.VMEM_SHARED((acc_rows, 16), jnp.float32),
            pltpu.VMEM_SHARED((acc_rows, 16), jnp.float32),
        ],
        compiler_params=pltpu.CompilerParams(use_tc_tiling_on_sc=False),
    )
    def k(src_hbm, dst_hbm, ones_hbm, zeros_hbm, osrc_hbm, odst_hbm,
          iblk_s, iblk_d, ones_v, zb, dsrc_sh, ddst_sh):
        c = lax.axis_index("c")
        s = lax.axis_index("s")
        w = s * NC + c
        pltpu.sync_copy(ones_hbm, ones_v)
        pltpu.sync_copy(zeros_hbm, zb)

        @pl.loop(0, n_zcopy)
        def _(i):
            r = s * rows_per_sub + i * ZR
            pltpu.sync_copy(zb, dsrc_sh.at[pl.ds(r, ZR)])
            pltpu.sync_copy(zb, ddst_sh.at[pl.ds(r, ZR)])

        plsc.subcore_barrier()

        pltpu.sync_copy(src_hbm.at[w], iblk_s)
        pltpu.sync_copy(dst_hbm.at[w], iblk_d)

        @pl.loop(0, nb)
        def _(j):
            pltpu.sync_copy(ones_v, dsrc_sh.at[iblk_s.at[j]], add=True)
            pltpu.sync_copy(ones_v, ddst_sh.at[iblk_d.at[j]], add=True)

        plsc.subcore_barrier()

        @pl.loop(0, n_zcopy)
        def _(i):
            r = s * rows_per_sub + i * ZR
            pltpu.sync_copy(dsrc_sh.at[pl.ds(r, ZR)], osrc_hbm.at[c, pl.ds(r, ZR)])
            pltpu.sync_copy(ddst_sh.at[pl.ds(r, ZR)], odst_hbm.at[c, pl.ds(r, ZR)])

    return k(src_idx, dst_idx, ones, zeros)


def _sc_segment_sum(h_r, src0, src1, dst_idx, zeros, acc_rows, nb):
    dh = h_r.shape[1]
    rows_per_sub = acc_rows // NS
    n_zcopy = rows_per_sub // ZR

    @functools.partial(
        pl.kernel,
        out_type=jax.ShapeDtypeStruct((NC, 2, acc_rows, dh), jnp.float32),
        mesh=_mesh(),
        scratch_types=[
            pltpu.VMEM((nb, B), jnp.int32),
            pltpu.VMEM((nb, B), jnp.int32),
            pltpu.VMEM((nb, B), jnp.int32),
            pltpu.VMEM((B, dh), jnp.float32),
            pltpu.VMEM((B, dh), jnp.float32),
            pltpu.VMEM((ZR, dh), jnp.float32),
            pltpu.VMEM_SHARED((acc_rows, dh), jnp.float32),
            pltpu.SemaphoreType.DMA,
            pltpu.SemaphoreType.DMA,
            pltpu.SemaphoreType.DMA,
            pltpu.SemaphoreType.DMA,
        ],
        compiler_params=pltpu.CompilerParams(use_tc_tiling_on_sc=False),
    )
    def k(h_hbm, s0_hbm, s1_hbm, dst_hbm, zeros_hbm, out_hbm, isv0, isv1,
          idblk, buf0, buf1, zb, acc_sh, sem0, sem1, sem2, sem3):
        c = lax.axis_index("c")
        s = lax.axis_index("s")
        w = s * NC + c
        pltpu.sync_copy(s0_hbm.at[w], isv0)
        pltpu.sync_copy(s1_hbm.at[w], isv1)
        pltpu.sync_copy(dst_hbm.at[w], idblk)
        pltpu.sync_copy(zeros_hbm, zb)

        for half, isv in ((0, isv0), (1, isv1)):
            @pl.loop(0, n_zcopy)
            def _(i):
                r = s * rows_per_sub + i * ZR
                pltpu.sync_copy(zb, acc_sh.at[pl.ds(r, ZR)])

            plsc.subcore_barrier()

            pltpu.async_copy(h_hbm.at[isv.at[0]], buf0, sem0)

            @pl.loop(0, nb // 2)
            def _(jj):
                j0 = 2 * jj
                j1 = j0 + 1
                pltpu.async_copy(h_hbm.at[isv.at[j1]], buf1, sem1)
                pltpu.make_async_copy(h_hbm.at[isv.at[j0]], buf0, sem0).wait()
                pltpu.sync_copy(buf0, acc_sh.at[idblk.at[j0]], add=True)

                @pl.when(jj < nb // 2 - 1)
                def _():
                    pltpu.async_copy(h_hbm.at[isv.at[j0 + 2]], buf0, sem0)

                pltpu.make_async_copy(h_hbm.at[isv.at[j1]], buf1, sem1).wait()
                pltpu.sync_copy(buf1, acc_sh.at[idblk.at[j1]], add=True)

            plsc.subcore_barrier()

            @pl.loop(0, n_zcopy)
            def _(i):
                r = s * rows_per_sub + i * ZR
                pltpu.sync_copy(acc_sh.at[pl.ds(r, ZR)],
                                out_hbm.at[c, half, pl.ds(r, ZR)])

    return k(h_r, src0, src1, dst_idx, zeros)


def _tc_matmul(x, w):
    def body(x_ref, w_ref, o_ref):
        o_ref[...] = jnp.dot(x_ref[...], w_ref[...],
                             preferred_element_type=jnp.float32)

    return pl.pallas_call(
        body,
        out_shape=jax.ShapeDtypeStruct((x.shape[0], w.shape[1]), jnp.float32),
    )(x, w)


def _tc_prep(dsrc_p, ddst_p, xw1, n):
    acc_rows = dsrc_p.shape[1]
    d_h = xw1.shape[1]

    def body(ds_ref, dd_ref, xw_ref, h_ref, nrm_ref):
        deg_out = jnp.sum(ds_ref[0], axis=1) + jnp.sum(ds_ref[1], axis=1)
        deg_in = jnp.sum(dd_ref[0], axis=1) + jnp.sum(dd_ref[1], axis=1)
        norm_src = jnp.where(deg_out > 0,
                             lax.rsqrt(jnp.maximum(deg_out, 1e-12)), 0.0)
        norm_dst = jnp.where(deg_in > 0,
                             lax.rsqrt(jnp.maximum(deg_in, 1e-12)), 0.0)
        nrm_ref[0, :] = norm_src
        nrm_ref[1, :] = norm_dst
        h_ref[...] = xw_ref[...] * norm_src[:n][:, None]

    return pl.pallas_call(
        body,
        out_shape=[
            jax.ShapeDtypeStruct((n, d_h), jnp.float32),
            jax.ShapeDtypeStruct((2, acc_rows), jnp.float32),
        ],
    )(dsrc_p, ddst_p, xw1)


def _tc_mid(m_parts, nrm, b1, w2, n):
    def body(m_ref, nrm_ref, b_ref, w_ref, z_ref, h2_ref):
        mp = m_ref[0] + m_ref[1]
        m = jnp.concatenate([mp[0, :n, :], mp[1, :n, :]], axis=1)
        z = jnp.maximum(m * nrm_ref[1, :n][:, None] + b_ref[...], 0.0)
        z_ref[...] = z
        h2 = z * nrm_ref[0, :n][:, None]
        h2_ref[...] = jnp.dot(h2, w_ref[...], preferred_element_type=jnp.float32)

    d = 2 * m_parts.shape[3]
    return pl.pallas_call(
        body,
        out_shape=[
            jax.ShapeDtypeStruct((n, d), jnp.float32),
            jax.ShapeDtypeStruct((n, w2.shape[1]), jnp.float32),
        ],
    )(m_parts, nrm, b1, w2)


def _tc_final(m_parts, nrm, b2, n):
    def body(m_ref, nrm_ref, b_ref, o_ref):
        mp = m_ref[0] + m_ref[1]
        m = jnp.concatenate([mp[0, :n, :], mp[1, :n, :]], axis=1)
        o_ref[...] = m * nrm_ref[1, :n][:, None] + b_ref[...]

    d = 2 * m_parts.shape[3]
    return pl.pallas_call(
        body,
        out_shape=jax.ShapeDtypeStruct((n, d), jnp.float32),
    )(m_parts, nrm, b2)


def kernel(in_feat, edge_index, W1, b1, W2, b2):
    n, d_in = in_feat.shape
    e = edge_index.shape[1]
    dh = d_in // 2

    nb = _cdiv(e, NW * B)
    nb += nb % 2
    e_pad = NW * nb * B
    acc_rows = _cdiv(n + 1, NS * ZR) * (NS * ZR)

    src = edge_index[0].astype(jnp.int32)
    dst = edge_index[1].astype(jnp.int32)
    pad = e_pad - e
    src_g = jnp.pad(src, (0, pad)).reshape(NW, nb, B)
    src0 = src_g * 2
    src1 = src_g * 2 + 1
    src_d = jnp.pad(src, (0, pad), constant_values=n).reshape(NW, nb, B)
    dst_d = jnp.pad(dst, (0, pad), constant_values=n).reshape(NW, nb, B)

    ones16 = jnp.zeros((B, 16), jnp.float32).at[:, 0].set(1.0)
    zeros16 = jnp.zeros((ZR, 16), jnp.float32)
    zeros_dh = jnp.zeros((ZR, dh), jnp.float32)

    b1r = b1.reshape(1, -1)
    b2r = b2.reshape(1, -1)

    dsrc_p, ddst_p = _sc_degrees(src_d, dst_d, ones16, zeros16, acc_rows, nb)
    xw1 = _tc_matmul(in_feat, W1)
    h1in, nrm = _tc_prep(dsrc_p, ddst_p, xw1, n)
    m1 = _sc_segment_sum(h1in.reshape(2 * n, dh), src0, src1, dst_d,
                         zeros_dh, acc_rows, nb)
    z, h2in = _tc_mid(m1, nrm, b1r, W2, n)
    m2 = _sc_segment_sum(h2in.reshape(2 * n, dh), src0, src1, dst_d,
                         zeros_dh, acc_rows, nb)
    h2 = _tc_final(m2, nrm, b2r, n)
    return (h2, z)


# speedup vs baseline: 1.1179x; 1.0965x over previous
import functools

import jax
import jax.numpy as jnp
from jax import lax
from jax.experimental import pallas as pl
from jax.experimental.pallas import tpu as pltpu
from jax.experimental.pallas import tpu_sc as plsc

NC, NS = 2, 16
NW = NC * NS
B = 128
ZR = 64


def _cdiv(a, b):
    return -(-a // b)


def _mesh():
    return plsc.VectorSubcoreMesh(core_axis_name="c", subcore_axis_name="s")


_SC_PARAMS = None


def _sc_degrees(src_idx, dst_idx, zeros, acc_rows, nb):
    out_t = [jax.ShapeDtypeStruct((NW, acc_rows), jnp.float32)] * 2

    @functools.partial(
        pl.kernel,
        out_type=out_t,
        mesh=_mesh(),
        scratch_types=[
            pltpu.VMEM((nb, B), jnp.int32),
            pltpu.VMEM((nb, B), jnp.int32),
            pltpu.VMEM((acc_rows,), jnp.float32),
            pltpu.VMEM((acc_rows,), jnp.float32),
        ],
        compiler_params=pltpu.CompilerParams(use_tc_tiling_on_sc=False,
                                             needs_layout_passes=False),
    )
    def k(src_hbm, dst_hbm, zeros_hbm, osrc_hbm, odst_hbm,
          isv, idv, dsrc_v, ddst_v):
        c = lax.axis_index("c")
        s = lax.axis_index("s")
        w = s * NC + c
        pltpu.sync_copy(src_hbm.at[w], isv)
        pltpu.sync_copy(dst_hbm.at[w], idv)
        pltpu.sync_copy(zeros_hbm, dsrc_v)
        pltpu.sync_copy(zeros_hbm, ddst_v)

        ones = jnp.ones((16,), jnp.float32)

        @pl.loop(0, nb)
        def _(j):
            for t in range(B // 16):
                plsc.addupdate_scatter(dsrc_v, [isv[j, pl.ds(t * 16, 16)]],
                                       ones)
                plsc.addupdate_scatter(ddst_v, [idv[j, pl.ds(t * 16, 16)]],
                                       ones)

        pltpu.sync_copy(dsrc_v, osrc_hbm.at[w])
        pltpu.sync_copy(ddst_v, odst_hbm.at[w])

    return k(src_idx, dst_idx, zeros)


def _sc_segment_sum(h_r, src0, src1, dst_idx, zeros, acc_rows, nb):
    dh = h_r.shape[1]
    rows_per_sub = acc_rows // NS
    n_zcopy = rows_per_sub // ZR

    @functools.partial(
        pl.kernel,
        out_type=jax.ShapeDtypeStruct((NC, 2, acc_rows, dh), jnp.float32),
        mesh=_mesh(),
        scratch_types=[
            pltpu.VMEM((nb, B), jnp.int32),
            pltpu.VMEM((nb, B), jnp.int32),
            pltpu.VMEM((nb, B), jnp.int32),
            pltpu.VMEM((B, dh), jnp.float32),
            pltpu.VMEM((B, dh), jnp.float32),
            pltpu.VMEM((ZR, dh), jnp.float32),
            pltpu.VMEM_SHARED((acc_rows, dh), jnp.float32),
            pltpu.SemaphoreType.DMA,
            pltpu.SemaphoreType.DMA,
            pltpu.SemaphoreType.DMA,
            pltpu.SemaphoreType.DMA,
        ],
        compiler_params=pltpu.CompilerParams(use_tc_tiling_on_sc=False),
    )
    def k(h_hbm, s0_hbm, s1_hbm, dst_hbm, zeros_hbm, out_hbm, isv0, isv1,
          idblk, buf0, buf1, zb, acc_sh, sem0, sem1, sem2, sem3):
        c = lax.axis_index("c")
        s = lax.axis_index("s")
        w = s * NC + c
        pltpu.sync_copy(s0_hbm.at[w], isv0)
        pltpu.sync_copy(s1_hbm.at[w], isv1)
        pltpu.sync_copy(dst_hbm.at[w], idblk)
        pltpu.sync_copy(zeros_hbm, zb)

        for half, isv in ((0, isv0), (1, isv1)):
            @pl.loop(0, n_zcopy)
            def _(i):
                r = s * rows_per_sub + i * ZR
                pltpu.sync_copy(zb, acc_sh.at[pl.ds(r, ZR)])

            plsc.subcore_barrier()

            pltpu.async_copy(h_hbm.at[isv.at[0]], buf0, sem0)

            @pl.loop(0, nb // 2)
            def _(jj):
                j0 = 2 * jj
                j1 = j0 + 1
                pltpu.async_copy(h_hbm.at[isv.at[j1]], buf1, sem1)
                pltpu.make_async_copy(h_hbm.at[isv.at[j0]], buf0, sem0).wait()
                pltpu.sync_copy(buf0, acc_sh.at[idblk.at[j0]], add=True)

                @pl.when(jj < nb // 2 - 1)
                def _():
                    pltpu.async_copy(h_hbm.at[isv.at[j0 + 2]], buf0, sem0)

                pltpu.make_async_copy(h_hbm.at[isv.at[j1]], buf1, sem1).wait()
                pltpu.sync_copy(buf1, acc_sh.at[idblk.at[j1]], add=True)

            plsc.subcore_barrier()

            @pl.loop(0, n_zcopy)
            def _(i):
                r = s * rows_per_sub + i * ZR
                pltpu.sync_copy(acc_sh.at[pl.ds(r, ZR)],
                                out_hbm.at[c, half, pl.ds(r, ZR)])

    return k(h_r, src0, src1, dst_idx, zeros)


def _tc_prep(dsrc_p, ddst_p, x, w1, n):
    acc_rows = dsrc_p.shape[1]
    d_h = w1.shape[1]

    def body(ds_ref, dd_ref, x_ref, w_ref, h_ref, nrm_ref):
        deg_out = jnp.sum(ds_ref[...], axis=0)
        deg_in = jnp.sum(dd_ref[...], axis=0)
        norm_src = jnp.where(deg_out > 0,
                             lax.rsqrt(jnp.maximum(deg_out, 1e-12)), 0.0)
        norm_dst = jnp.where(deg_in > 0,
                             lax.rsqrt(jnp.maximum(deg_in, 1e-12)), 0.0)
        nrm_ref[0, :] = norm_src
        nrm_ref[1, :] = norm_dst
        h = x_ref[...] * norm_src[:n][:, None]
        h_ref[...] = jnp.dot(h, w_ref[...], preferred_element_type=jnp.float32)

    return pl.pallas_call(
        body,
        out_shape=[
            jax.ShapeDtypeStruct((n, d_h), jnp.float32),
            jax.ShapeDtypeStruct((2, acc_rows), jnp.float32),
        ],
    )(dsrc_p, ddst_p, x, w1)


def _tc_mid(m_parts, nrm, b1, w2, n):
    def body(m_ref, nrm_ref, b_ref, w_ref, z_ref, h2_ref):
        mp = m_ref[0] + m_ref[1]
        m = jnp.concatenate([mp[0, :n, :], mp[1, :n, :]], axis=1)
        z = jnp.maximum(m * nrm_ref[1, :n][:, None] + b_ref[...], 0.0)
        z_ref[...] = z
        h2 = z * nrm_ref[0, :n][:, None]
        h2_ref[...] = jnp.dot(h2, w_ref[...], preferred_element_type=jnp.float32)

    d = 2 * m_parts.shape[3]
    return pl.pallas_call(
        body,
        out_shape=[
            jax.ShapeDtypeStruct((n, d), jnp.float32),
            jax.ShapeDtypeStruct((n, w2.shape[1]), jnp.float32),
        ],
    )(m_parts, nrm, b1, w2)


def _tc_final(m_parts, nrm, b2, n):
    def body(m_ref, nrm_ref, b_ref, o_ref):
        mp = m_ref[0] + m_ref[1]
        m = jnp.concatenate([mp[0, :n, :], mp[1, :n, :]], axis=1)
        o_ref[...] = m * nrm_ref[1, :n][:, None] + b_ref[...]

    d = 2 * m_parts.shape[3]
    return pl.pallas_call(
        body,
        out_shape=jax.ShapeDtypeStruct((n, d), jnp.float32),
    )(m_parts, nrm, b2)


def kernel(in_feat, edge_index, W1, b1, W2, b2):
    n, d_in = in_feat.shape
    e = edge_index.shape[1]
    dh = d_in // 2

    nb = _cdiv(e, NW * B)
    nb += nb % 2
    e_pad = NW * nb * B
    acc_rows = _cdiv(n + 1, NS * ZR) * (NS * ZR)

    src = edge_index[0].astype(jnp.int32)
    dst = edge_index[1].astype(jnp.int32)
    pad = e_pad - e
    src_g = jnp.pad(src, (0, pad)).reshape(NW, nb, B)
    src0 = src_g * 2
    src1 = src_g * 2 + 1
    src_d = jnp.pad(src, (0, pad), constant_values=n).reshape(NW, nb, B)
    dst_d = jnp.pad(dst, (0, pad), constant_values=n).reshape(NW, nb, B)

    zeros_acc = jnp.zeros((acc_rows,), jnp.float32)
    zeros_dh = jnp.zeros((ZR, dh), jnp.float32)

    b1r = b1.reshape(1, -1)
    b2r = b2.reshape(1, -1)

    dsrc_p, ddst_p = _sc_degrees(src_d, dst_d, zeros_acc, acc_rows, nb)
    h1in, nrm = _tc_prep(dsrc_p, ddst_p, in_feat, W1, n)
    m1 = _sc_segment_sum(h1in.reshape(2 * n, dh), src0, src1, dst_d,
                         zeros_dh, acc_rows, nb)
    z, h2in = _tc_mid(m1, nrm, b1r, W2, n)
    m2 = _sc_segment_sum(h2in.reshape(2 * n, dh), src0, src1, dst_d,
                         zeros_dh, acc_rows, nb)
    h2 = _tc_final(m2, nrm, b2r, n)
    return (h2, z)
